# baseline (speedup 1.0000x reference)
import functools

import jax
import jax.numpy as jnp
from jax import lax
from jax.experimental import pallas as pl
from jax.experimental.pallas import tpu as pltpu
from jax.experimental.pallas import tpu_sc as plsc

_L = 16


@functools.partial(jax.jit, static_argnums=(0, 1, 2))
def _pos_embed_sc(B, H, W, row_embed, col_embed):
    F = row_embed.shape[1]
    C = 2 * F
    info = plsc.get_sparse_core_info()
    NS = info.num_subcores
    HPW = H // NS
    assert H % NS == 0

    mesh = plsc.VectorSubcoreMesh(
        core_axis_name="c", subcore_axis_name="s", num_cores=1)

    @functools.partial(
        pl.kernel,
        mesh=mesh,
        compiler_params=pltpu.CompilerParams(
            use_tc_tiling_on_sc=True, needs_layout_passes=False),
        out_type=jax.ShapeDtypeStruct((B, H, W, C), jnp.float32),
        scratch_types=[
            pltpu.VMEM((HPW, W, C), jnp.float32),
            pltpu.VMEM((HPW, F), jnp.float32),
            pltpu.SemaphoreType.DMA,
            pltpu.SemaphoreType.DMA,
        ],
    )
    def k(row_hbm, col_hbm, out_hbm, plane_v, rowbuf_v, sstage, sout):
        sid = lax.axis_index("s")
        h0 = sid * HPW
        cstages = [
            pltpu.async_copy(
                col_hbm.at[pl.ds(0, W)], plane_v.at[r, :, pl.ds(0, F)], sstage)
            for r in range(HPW)
        ]
        pltpu.sync_copy(row_hbm.at[pl.ds(h0, HPW)], rowbuf_v)
        vs = [[rowbuf_v[r, pl.ds(j * _L, _L)] for j in range(F // _L)]
              for r in range(HPW)]
        def repl(w, carry):
            for r in range(HPW):
                for j in range(F // _L):
                    plane_v[r, w, pl.ds(F + j * _L, _L)] = vs[r][j]
            return carry
        lax.fori_loop(0, W, repl, 0)
        for hd in cstages:
            hd.wait()
        handles = [
            pltpu.async_copy(plane_v.at[r], out_hbm.at[b, h0 + r], sout)
            for r in range(HPW) for b in range(B)
        ]
        for hd in handles:
            hd.wait()

    return k(row_embed, col_embed)


def kernel(x, row_embed, col_embed):
    B = x.shape[0]
    H, W = x.shape[-2], x.shape[-1]
    out_nhwc = _pos_embed_sc(B, H, W, row_embed, col_embed)
    return jnp.transpose(out_nhwc, (0, 3, 1, 2))


# speedup vs baseline: 1.0644x; 1.0644x over previous
import functools

import jax
import jax.numpy as jnp
from jax import lax
from jax.experimental import pallas as pl
from jax.experimental.pallas import tpu as pltpu
from jax.experimental.pallas import tpu_sc as plsc

_L = 16


@functools.partial(jax.jit, static_argnums=(0, 1, 2))
def _pos_embed_sc(B, H, W, row_embed, col_embed):
    F = row_embed.shape[1]
    C = 2 * F
    info = plsc.get_sparse_core_info()
    NS = info.num_subcores
    HPW = H // NS
    assert H % NS == 0

    mesh = plsc.VectorSubcoreMesh(
        core_axis_name="c", subcore_axis_name="s", num_cores=1)

    @functools.partial(
        pl.kernel,
        mesh=mesh,
        compiler_params=pltpu.CompilerParams(
            use_tc_tiling_on_sc=True, needs_layout_passes=False),
        out_type=jax.ShapeDtypeStruct((B, H, W, C), jnp.float32),
        scratch_types=[
            pltpu.VMEM((W, F), jnp.float32),
            pltpu.VMEM((HPW, W, F), jnp.float32),
            pltpu.VMEM((HPW, F), jnp.float32),
            pltpu.SemaphoreType.DMA,
            pltpu.SemaphoreType.DMA,
        ],
    )
    def k(row_hbm, col_hbm, out_hbm, colp_v, rowp_v, rowbuf_v, sstage, sout):
        sid = lax.axis_index("s")
        h0 = sid * HPW
        cstage = pltpu.async_copy(col_hbm.at[pl.ds(0, W)], colp_v, sstage)
        pltpu.sync_copy(row_hbm.at[pl.ds(h0, HPW)], rowbuf_v)
        vs = [[rowbuf_v[r, pl.ds(j * _L, _L)] for j in range(F // _L)]
              for r in range(HPW)]
        def repl(w, carry):
            for r in range(HPW):
                for j in range(F // _L):
                    rowp_v[r, w, pl.ds(j * _L, _L)] = vs[r][j]
            return carry
        lax.fori_loop(0, W, repl, 0)
        handles = [
            pltpu.async_copy(
                rowp_v.at[r], out_hbm.at[b, h0 + r, :, pl.ds(F, F)], sout)
            for r in range(HPW) for b in range(B)
        ]
        cstage.wait()
        for r in range(HPW):
            for b in range(B):
                handles.append(pltpu.async_copy(
                    colp_v, out_hbm.at[b, h0 + r, :, pl.ds(0, F)], sout))
        for hd in handles:
            hd.wait()

    return k(row_embed, col_embed)


def kernel(x, row_embed, col_embed):
    B = x.shape[0]
    H, W = x.shape[-2], x.shape[-1]
    out_nhwc = _pos_embed_sc(B, H, W, row_embed, col_embed)
    return jnp.transpose(out_nhwc, (0, 3, 1, 2))
